# baseline (speedup 1.0000x reference)
import functools

import jax
import jax.numpy as jnp
from jax import lax
from jax.experimental import pallas as pl
from jax.experimental.pallas import tpu as pltpu
from jax.experimental.pallas import tpu_sc as plsc

D_MODEL = 64
SCALE = 8.0
VOCAB = 1000000
NC, NS, L = 2, 16, 16
NW = NC * NS
VT = (VOCAB + 127) // 128
S_ROWS = VT * 128
K1_TPW = (VT + NW - 1) // NW

B0, B1 = 4096, 200
NB0T = B0 // 128
PANELS = B1 * NB0T
K2_PPW = PANELS // NW


def _mesh():
    return plsc.VectorSubcoreMesh(
        core_axis_name="c", subcore_axis_name="s",
        num_cores=NC, num_subcores=NS)


def _wid():
    return lax.axis_index("s") * NC + lax.axis_index("c")


def _segs(n):
    return [k * L + lax.iota(jnp.int32, L) for k in range(n)]


@functools.partial(
    pl.kernel,
    out_type=jax.ShapeDtypeStruct((S_ROWS, 128), jnp.float32),
    mesh=_mesh(),
    scratch_types=[
        pltpu.VMEM((2, 64, 128), jnp.float32),
        pltpu.VMEM((2, 128, 129), jnp.float32),
        pltpu.SemaphoreType.DMA((2,)),
        pltpu.SemaphoreType.DMA((2,)),
    ],
    compiler_params=pltpu.CompilerParams(
        use_tc_tiling_on_sc=True, needs_layout_passes=False),
)
def _relayout(tt, s, tbuf, rbuf, isem, osem):
    w = _wid()
    start = w * K1_TPW
    n_my = jnp.minimum(K1_TPW, VT - start)
    segs = _segs(8)

    def in_copy(vt, b):
        return pltpu.make_async_copy(
            tt.at[:, pl.ds(vt * 128, 128)], tbuf.at[b], isem.at[b])

    def out_copy(vt, b):
        return pltpu.make_async_copy(
            rbuf.at[b, :, pl.ds(0, 128)], s.at[pl.ds(vt * 128, 128)],
            osem.at[b])

    def transpose(b):
        @plsc.parallel_loop(0, D_MODEL, unroll=8, carry=jnp.int32(0))
        def _(d, c):
            colv = jnp.full((L,), d, jnp.int32)
            for k in range(8):
                v = tbuf[b, d, pl.ds(k * L, L)]
                rbuf[b, d, pl.ds(k * L, L)] = v * SCALE
            return c

    in_copy(start, 0).start()

    def body(t, carry):
        b = lax.rem(t, 2)
        vt = start + t

        @pl.when(t + 1 < n_my)
        def _():
            in_copy(vt + 1, 1 - b).start()

        in_copy(vt, b).wait()

        @pl.when(t >= 2)
        def _():
            out_copy(vt - 2, b).wait()

        transpose(b)
        out_copy(vt, b).start()
        return carry

    lax.fori_loop(0, n_my, body, 0)
    out_copy(start + n_my - 2, lax.rem(n_my - 2, 2)).wait()
    out_copy(start + n_my - 1, lax.rem(n_my - 1, 2)).wait()


@functools.partial(
    pl.kernel,
    out_type=jax.ShapeDtypeStruct((B1, 8, NB0T, 8, 128), jnp.float32),
    mesh=_mesh(),
    scratch_types=[
        pltpu.VMEM((2, 128), jnp.int32),
        pltpu.VMEM((2, 128, 128), jnp.float32),
        pltpu.VMEM((2, 64, 129), jnp.float32),
        pltpu.SemaphoreType.DMA((2,)),
        pltpu.SemaphoreType.DMA((2,)),
        pltpu.SemaphoreType.DMA((2,)),
    ],
    compiler_params=pltpu.CompilerParams(
        use_tc_tiling_on_sc=True, needs_layout_passes=False),
)
def _gather(xt, s, out, idxv, dst, panel, isem, gsem, osem):
    w = _wid()
    p0 = w * K2_PPW
    segs = _segs(8)

    def coords(p):
        return p // NB0T, lax.rem(p, NB0T)

    def idx_copy(p, b):
        b1, b0t = coords(p)
        return pltpu.make_async_copy(
            xt.at[b1, pl.ds(b0t * 128, 128)], idxv.at[b], isem.at[b])

    def gather_copy(b):
        return pltpu.make_async_copy(s.at[idxv.at[b]], dst.at[b], gsem.at[b])

    def out_copy(p, b):
        b1, b0t = coords(p)
        for dt in range(8):
            yield pltpu.make_async_copy(
                panel.at[b, pl.ds(dt * 8, 8), pl.ds(0, 128)],
                out.at[b1, dt, b0t], osem.at[b])

    def out_start(p, b):
        for c in out_copy(p, b):
            c.start()

    def out_wait(p, b):
        for c in out_copy(p, b):
            c.wait()

    def transpose(b):
        @plsc.parallel_loop(0, 128, unroll=8, carry=jnp.int32(0))
        def _(j, c):
            colv = jnp.full((L,), j, jnp.int32)
            for k in range(4):
                v = dst[b, j, pl.ds(k * L, L)]
                plsc.store_scatter(panel.at[b], [segs[k], colv], v)
            return c

    pltpu.sync_copy(xt.at[coords(p0)[0], pl.ds(coords(p0)[1] * 128, 128)],
                    idxv.at[0])
    gather_copy(0).start()
    idx_copy(p0 + 1, 1).start()

    def body(t, carry):
        b = lax.rem(t, 2)
        p = p0 + t

        gather_copy(b).wait()

        @pl.when(t + 1 < K2_PPW)
        def _():
            idx_copy(p + 1, 1 - b).wait()
            gather_copy(1 - b).start()

        @pl.when(t + 2 < K2_PPW)
        def _():
            idx_copy(p + 2, b).start()

        @pl.when(t >= 2)
        def _():
            out_wait(p - 2, b)

        transpose(b)
        out_start(p, b)
        return carry

    lax.fori_loop(0, K2_PPW, body, 0)
    out_wait(p0 + K2_PPW - 2, (K2_PPW - 2) % 2)
    out_wait(p0 + K2_PPW - 1, (K2_PPW - 1) % 2)


def kernel(x, lut_weight):
    tt = lut_weight.T
    staged = _relayout(tt)
    xt = x.astype(jnp.int32).T
    out5 = _gather(xt, staged)
    out5 = out5.reshape(B1, 8, NB0T, 8, 128)
    return jnp.transpose(out5, (2, 4, 0, 1, 3)).reshape(B0, B1, D_MODEL)


# speedup vs baseline: 1.0583x; 1.0583x over previous
import functools

import jax
import jax.numpy as jnp
from jax import lax
from jax.experimental import pallas as pl
from jax.experimental.pallas import tpu as pltpu
from jax.experimental.pallas import tpu_sc as plsc

D_MODEL = 64
SCALE = 8.0
VOCAB = 1000000
NC, NS = 2, 16
NW = NC * NS

B0, B1 = 4096, 200
B = B0 * B1
ROWS_PW = B // NW
CHUNK = 128
NCHUNK = ROWS_PW // CHUNK

VCHUNK = 4096


def _stage_body(a_ref, o_ref):
    b = a_ref[...].T * SCALE
    o_ref[...] = jnp.concatenate([b, b], axis=1)


@jax.jit
def _stage(tt):
    return pl.pallas_call(
        _stage_body,
        grid=(pl.cdiv(VOCAB, VCHUNK),),
        in_specs=[pl.BlockSpec((D_MODEL, VCHUNK), lambda c: (0, c))],
        out_specs=pl.BlockSpec((VCHUNK, 128), lambda c: (c, 0)),
        out_shape=jax.ShapeDtypeStruct((VOCAB, 128), jnp.float32),
    )(tt)


@functools.partial(
    pl.kernel,
    out_type=jax.ShapeDtypeStruct((B, 128), jnp.float32),
    mesh=plsc.VectorSubcoreMesh(
        core_axis_name="c", subcore_axis_name="s",
        num_cores=NC, num_subcores=NS),
    scratch_types=[
        pltpu.VMEM((2, CHUNK), jnp.int32),
        pltpu.VMEM((2, CHUNK, 128), jnp.float32),
        pltpu.SemaphoreType.DMA((2,)),
        pltpu.SemaphoreType.DMA((2,)),
        pltpu.SemaphoreType.DMA((2,)),
    ],
    compiler_params=pltpu.CompilerParams(
        use_tc_tiling_on_sc=True, needs_layout_passes=False),
)
def _gather(xflat, s, out, idxv, dst, isem, gsem, osem):
    w = lax.axis_index("s") * NC + lax.axis_index("c")
    base = w * ROWS_PW

    def idx_copy(t, b):
        return pltpu.make_async_copy(
            xflat.at[pl.ds(base + t * CHUNK, CHUNK)], idxv.at[b], isem.at[b])

    def gather_copy(b):
        return pltpu.make_async_copy(s.at[idxv.at[b]], dst.at[b], gsem.at[b])

    def out_copy(t, b):
        return pltpu.make_async_copy(
            dst.at[b], out.at[pl.ds(base + t * CHUNK, CHUNK)], osem.at[b])

    idx_copy(0, 0).start()
    idx_copy(0, 0).wait()
    gather_copy(0).start()
    idx_copy(1, 1).start()

    def body(t, carry):
        b = lax.rem(t, 2)

        gather_copy(b).wait()

        @pl.when(t + 1 < NCHUNK)
        def _():
            idx_copy(t + 1, 1 - b).wait()

            @pl.when(t >= 1)
            def _():
                out_copy(t - 1, 1 - b).wait()

            gather_copy(1 - b).start()

        @pl.when(t + 2 < NCHUNK)
        def _():
            idx_copy(t + 2, b).start()

        out_copy(t, b).start()
        return carry

    lax.fori_loop(0, NCHUNK, body, 0)
    out_copy(NCHUNK - 2, (NCHUNK - 2) % 2).wait()
    out_copy(NCHUNK - 1, (NCHUNK - 1) % 2).wait()


def kernel(x, lut_weight):
    tt = lut_weight.T
    staged = _stage(tt)
    xflat = x.astype(jnp.int32).reshape(B)
    out = _gather(xflat, staged)
    return out.reshape(B0, B1, 128)[:, :, :D_MODEL]


# speedup vs baseline: 1.1582x; 1.0945x over previous
import functools

import jax
import jax.numpy as jnp
from jax import lax
from jax.experimental import pallas as pl
from jax.experimental.pallas import tpu as pltpu
from jax.experimental.pallas import tpu_sc as plsc

D_MODEL = 64
SCALE = 8.0
VOCAB = 1000000
NC, NS = 2, 16
NW = NC * NS

B0, B1 = 4096, 200
B = B0 * B1
ROWS_PW = B // NW
CHUNK = 128
NCHUNK = ROWS_PW // CHUNK

VCHUNK = 4096


def _stage_body(a_ref, o_ref):
    b = a_ref[...].T * SCALE
    o_ref[...] = jnp.concatenate([b, b], axis=1)


@jax.jit
def _stage(tt):
    return pl.pallas_call(
        _stage_body,
        grid=(pl.cdiv(VOCAB, VCHUNK),),
        in_specs=[pl.BlockSpec((D_MODEL, VCHUNK), lambda c: (0, c))],
        out_specs=pl.BlockSpec((VCHUNK, 128), lambda c: (c, 0)),
        out_shape=jax.ShapeDtypeStruct((VOCAB, 128), jnp.float32),
    )(tt)


@functools.partial(
    pl.kernel,
    out_type=jax.ShapeDtypeStruct((B, 128), jnp.float32),
    mesh=plsc.VectorSubcoreMesh(
        core_axis_name="c", subcore_axis_name="s",
        num_cores=NC, num_subcores=NS),
    scratch_types=[
        pltpu.VMEM((3, CHUNK), jnp.int32),
        pltpu.VMEM((3, CHUNK, 128), jnp.float32),
        pltpu.SemaphoreType.DMA((3,)),
        pltpu.SemaphoreType.DMA((3,)),
        pltpu.SemaphoreType.DMA((3,)),
    ],
    compiler_params=pltpu.CompilerParams(
        use_tc_tiling_on_sc=True, needs_layout_passes=False),
)
def _gather(xflat, s, out, idxv, dst, isem, gsem, osem):
    w = lax.axis_index("s") * NC + lax.axis_index("c")
    base = w * ROWS_PW

    def idx_copy(t, b):
        return pltpu.make_async_copy(
            xflat.at[pl.ds(base + t * CHUNK, CHUNK)], idxv.at[b], isem.at[b])

    def gather_copy(b):
        return pltpu.make_async_copy(s.at[idxv.at[b]], dst.at[b], gsem.at[b])

    def out_copy(t, b):
        return pltpu.make_async_copy(
            dst.at[b], out.at[pl.ds(base + t * CHUNK, CHUNK)], osem.at[b])

    idx_copy(0, 0).start()
    idx_copy(1, 1).start()
    idx_copy(0, 0).wait()
    gather_copy(0).start()
    idx_copy(1, 1).wait()
    gather_copy(1).start()
    idx_copy(2, 2).start()

    def body(t, carry):
        b = lax.rem(t, 3)
        b1 = lax.rem(t + 1, 3)
        b2 = lax.rem(t + 2, 3)

        gather_copy(b).wait()

        @pl.when(t + 2 < NCHUNK)
        def _():
            idx_copy(t + 2, b2).wait()

            @pl.when(t >= 1)
            def _():
                out_copy(t - 1, b2).wait()

            gather_copy(b2).start()

        @pl.when(t + 3 < NCHUNK)
        def _():
            idx_copy(t + 3, b).start()

        out_copy(t, b).start()
        return carry

    lax.fori_loop(0, NCHUNK, body, 0)
    out_copy(NCHUNK - 2, (NCHUNK - 2) % 3).wait()
    out_copy(NCHUNK - 1, (NCHUNK - 1) % 3).wait()


def kernel(x, lut_weight):
    tt = lut_weight.T
    staged = _stage(tt)
    xflat = x.astype(jnp.int32).reshape(B)
    out = _gather(xflat, staged)
    return out.reshape(B0, B1, 128)[:, :, :D_MODEL]


# speedup vs baseline: 1.2461x; 1.0759x over previous
import functools

import jax
import jax.numpy as jnp
from jax import lax
from jax.experimental import pallas as pl
from jax.experimental.pallas import tpu as pltpu
from jax.experimental.pallas import tpu_sc as plsc

D_MODEL = 64
SCALE = 8.0
VOCAB = 1000000
NC, NS = 2, 16
NW = NC * NS

B0, B1 = 4096, 200
B = B0 * B1
ROWS_PW = B // NW
CHUNK = 128
NCHUNK = ROWS_PW // CHUNK

VCHUNK = 4096


def _stage_body(a_ref, o_ref):
    b = a_ref[...].T * SCALE
    o_ref[...] = jnp.concatenate(
        [b[: VCHUNK // 2], b[VCHUNK // 2:]], axis=1)


@jax.jit
def _stage(tt):
    return pl.pallas_call(
        _stage_body,
        grid=(pl.cdiv(VOCAB, VCHUNK),),
        in_specs=[pl.BlockSpec((D_MODEL, VCHUNK), lambda c: (0, c))],
        out_specs=pl.BlockSpec((VCHUNK // 2, 128), lambda c: (c, 0)),
        out_shape=jax.ShapeDtypeStruct(
            (pl.cdiv(VOCAB, VCHUNK) * (VCHUNK // 2), 128), jnp.float32),
    )(tt)


@functools.partial(
    pl.kernel,
    out_type=jax.ShapeDtypeStruct((B, 128), jnp.float32),
    mesh=plsc.VectorSubcoreMesh(
        core_axis_name="c", subcore_axis_name="s",
        num_cores=NC, num_subcores=NS),
    scratch_types=[
        pltpu.VMEM((3, CHUNK), jnp.int32),
        pltpu.VMEM((3, CHUNK, 128), jnp.float32),
        pltpu.VMEM((3, CHUNK, 128), jnp.float32),
        pltpu.VMEM((3, CHUNK), jnp.int32),
        pltpu.SemaphoreType.DMA((3,)),
        pltpu.SemaphoreType.DMA((3,)),
        pltpu.SemaphoreType.DMA((3,)),
    ],
    compiler_params=pltpu.CompilerParams(
        use_tc_tiling_on_sc=True, needs_layout_passes=False),
)
def _gather(xflat, s, out, idxv, dst, packed, idx2v, isem, gsem, osem):
    w = lax.axis_index("s") * NC + lax.axis_index("c")
    base = w * ROWS_PW

    def idx_copy(t, b):
        return pltpu.make_async_copy(
            xflat.at[pl.ds(base + t * CHUNK, CHUNK)], idxv.at[b], isem.at[b])

    def gather_copy(b):
        return pltpu.make_async_copy(s.at[idx2v.at[b]], dst.at[b], gsem.at[b])

    def out_copy(t, b):
        return pltpu.make_async_copy(
            packed.at[b], out.at[pl.ds(base + t * CHUNK, CHUNK)], osem.at[b])

    def prep(b):
        for k in range(CHUNK // 16):
            sl = pl.ds(k * 16, 16)
            v = idxv[b, sl]
            idx2v[b, sl] = (
                jax.lax.shift_left(
                    jax.lax.shift_right_logical(v, 12), 11)
                + (v & (VCHUNK // 2 - 1)))

    def repack(b):
        @plsc.parallel_loop(0, CHUNK, unroll=4, carry=jnp.int32(0))
        def _(j, c):
            seg = idxv[b, pl.ds(lax.div(j, 16) * 16, 16)]
            lane = jnp.full((16,), lax.rem(j, 16), jnp.int32)
            odd = (jax.lax.shift_right_logical(seg[lane], 11) & 1) == 1
            for k in range(4):
                lo = dst[b, j, pl.ds(k * 16, 16)]
                hi = dst[b, j, pl.ds(D_MODEL + k * 16, 16)]
                packed[b, j, pl.ds(k * 16, 16)] = jnp.where(odd, hi, lo)
            return c

    idx_copy(0, 0).start()
    idx_copy(1, 1).start()
    idx_copy(0, 0).wait()
    prep(0)
    gather_copy(0).start()
    idx_copy(1, 1).wait()
    prep(1)
    gather_copy(1).start()
    idx_copy(2, 2).start()

    def body(t, carry):
        b = lax.rem(t, 3)
        b2 = lax.rem(t + 2, 3)

        gather_copy(b).wait()

        @pl.when(t + 2 < NCHUNK)
        def _():
            idx_copy(t + 2, b2).wait()
            prep(b2)
            gather_copy(b2).start()

        @pl.when(t + 3 < NCHUNK)
        def _():
            idx_copy(t + 3, b).start()

        @pl.when(t >= 3)
        def _():
            out_copy(t - 3, b).wait()

        repack(b)
        out_copy(t, b).start()
        return carry

    lax.fori_loop(0, NCHUNK, body, 0)
    out_copy(NCHUNK - 3, (NCHUNK - 3) % 3).wait()
    out_copy(NCHUNK - 2, (NCHUNK - 2) % 3).wait()
    out_copy(NCHUNK - 1, (NCHUNK - 1) % 3).wait()


def kernel(x, lut_weight):
    tt = lut_weight.T
    staged = _stage(tt)
    xflat = x.astype(jnp.int32).reshape(B)
    out = _gather(xflat, staged)
    return out.reshape(B0, B1, 128)[:, :, :D_MODEL]
